# Initial kernel scaffold; baseline (speedup 1.0000x reference)
import jax
import jax.numpy as jnp
from jax.experimental import pallas as pl


def kernel(x, batch, W_g1, b_g1, W_g2, b_g2, W_l, b_l):
    raise NotImplementedError("write your pallas kernel here")


# speedup vs baseline: 6.8427x; 6.8427x over previous
import functools

import jax
import jax.numpy as jnp
from jax import lax
from jax.experimental import pallas as pl
from jax.experimental.pallas import tpu as pltpu
from jax.experimental.pallas import tpu_sc as plsc

N = 100000
D = 128
H = 64
G = 512

NT = 32
SPT = G // NT
CB = 256
CBP = CB + 8
CG = 1024
GB = CBP + 8

_R = 2000


def _gelu(v):
    return v * 0.5 * (1.0 + lax.erf(v * (2.0 ** -0.5)))


def _dense_body(x_ref, wg1_ref, bg1_ref, wg2_ref, bg2_ref, wl_ref, bl_ref,
                gate_ref, local_ref):
    x = x_ref[...]
    h = jnp.dot(x, wg1_ref[...], preferred_element_type=jnp.float32)
    h = _gelu(h + bg1_ref[...])
    gate_ref[...] = (jnp.sum(h * wg2_ref[...], axis=1, keepdims=True)
                     + bg2_ref[...])
    loc = jnp.dot(x, wl_ref[...], preferred_element_type=jnp.float32)
    local_ref[...] = _gelu(loc + bl_ref[...])


def _dense(x, W_g1, b_g1, W_g2, b_g2, W_l, b_l):
    return pl.pallas_call(
        _dense_body,
        grid=(N // _R,),
        in_specs=[
            pl.BlockSpec((_R, D), lambda i: (i, 0)),
            pl.BlockSpec((D, H), lambda i: (0, 0)),
            pl.BlockSpec((1, H), lambda i: (0, 0)),
            pl.BlockSpec((1, H), lambda i: (0, 0)),
            pl.BlockSpec((1, 1), lambda i: (0, 0)),
            pl.BlockSpec((D, H), lambda i: (0, 0)),
            pl.BlockSpec((1, H), lambda i: (0, 0)),
        ],
        out_specs=[
            pl.BlockSpec((_R, 1), lambda i: (i, 0)),
            pl.BlockSpec((_R, H), lambda i: (i, 0)),
        ],
        out_shape=[
            jax.ShapeDtypeStruct((N, 1), jnp.float32),
            jax.ShapeDtypeStruct((N, H), jnp.float32),
        ],
    )(x, W_g1, b_g1.reshape(1, H), W_g2.reshape(1, H), b_g2.reshape(1, 1),
      W_l, b_l.reshape(1, H))


def _tile_id():
    return lax.axis_index("c") * 16 + lax.axis_index("s")


def _sc_body(x_hbm, local_hbm, gate_hbm, bnd_hbm, out_hbm,
             bnd_v, gbufA, gbufB, ebuf, xbuf, lbuf, outbuf):
    tid = _tile_id()
    base_seg = tid * SPT
    pltpu.sync_copy(bnd_hbm.at[pl.ds(base_seg, 40)], bnd_v)

    lane = lax.iota(jnp.int32, 16)

    def seg_body(s, _):
        r0 = bnd_v[pl.ds(s, 16)][0]
        r1 = bnd_v[pl.ds(s + 1, 16)][0]

        a0 = (r0 // 8) * 8
        nA = (r1 - a0 + CG - 1) // CG

        def passA(k, m):
            cs = jnp.minimum(a0 + k * CG, N - CG)
            pltpu.sync_copy(gate_hbm.at[pl.ds(cs, CG)], gbufA)
            for j in range(CG // 16):
                idx = cs + j * 16 + lane
                valid = (idx >= r0) & (idx < r1)
                g = gbufA[pl.ds(j * 16, 16)]
                m = jnp.maximum(m, jnp.where(valid, g, -jnp.inf))
            return m

        m = lax.fori_loop(0, nA, passA, jnp.full((16,), -jnp.inf, jnp.float32))
        gmax = m[0]
        for i in range(1, 16):
            gmax = jnp.maximum(gmax, m[i])

        nB = (r1 - r0 + CB - 1) // CB
        z = jnp.zeros((16,), jnp.float32)
        ninf = jnp.full((16,), -jnp.inf, jnp.float32)
        init = ((z,) * 8, (ninf,) * 8, (z,) * 8, (z,) * 4,
                jnp.float32(0.0))

        def passB(k, carry):
            lo = r0 + k * CB
            hi = jnp.minimum(lo + CB, r1)
            rs = jnp.minimum((lo // 8) * 8, N - CBP)
            pltpu.sync_copy(x_hbm.at[pl.ds(rs, CBP)], xbuf)
            pltpu.sync_copy(local_hbm.at[pl.ds(rs, CBP)], lbuf)
            ga = jnp.minimum(rs, N - GB)
            pltpu.sync_copy(gate_hbm.at[pl.ds(ga, GB)], gbufB)
            for j in range(GB // 16):
                ebuf[pl.ds(j * 16, 16)] = jnp.exp(
                    gbufB[pl.ds(j * 16, 16)] - gmax)

            def row(i, c):
                su, mx, at, ls, es = c
                off = i - rs
                e = ebuf[pl.ds(i - ga, 16)][0]
                su2, mx2, at2, ls2 = [], [], [], []
                for v in range(8):
                    xv = xbuf[off, pl.ds(v * 16, 16)]
                    su2.append(su[v] + xv)
                    mx2.append(jnp.maximum(mx[v], xv))
                    at2.append(at[v] + e * xv)
                for v in range(4):
                    ls2.append(ls[v] + lbuf[off, pl.ds(v * 16, 16)])
                return (tuple(su2), tuple(mx2), tuple(at2), tuple(ls2),
                        es + e)

            return lax.fori_loop(lo, hi, row, carry)

        su, mx, at, ls, esum = lax.fori_loop(0, nB, passB, init)

        cnt = r1 - r0
        cv = jnp.full((16,), jnp.maximum(cnt, 1).astype(jnp.float32))
        ev = jnp.full((16,), jnp.where(cnt == 0, jnp.float32(1.0), esum))
        for v in range(8):
            outbuf[s, pl.ds(v * 16, 16)] = su[v] / cv
            outbuf[s, pl.ds(D + v * 16, 16)] = mx[v]
            outbuf[s, pl.ds(2 * D + v * 16, 16)] = at[v] / ev
        for v in range(4):
            outbuf[s, pl.ds(3 * D + v * 16, 16)] = ls[v] / cv
        return 0

    lax.fori_loop(0, SPT, seg_body, 0)
    pltpu.sync_copy(outbuf, out_hbm.at[pl.ds(base_seg, SPT)])


def _segment_pool(x, local, gate, bnd):
    mesh = plsc.VectorSubcoreMesh(core_axis_name="c", subcore_axis_name="s")
    f = pl.kernel(
        _sc_body,
        out_type=jax.ShapeDtypeStruct((G, 3 * D + H), jnp.float32),
        mesh=mesh,
        scratch_types=[
            pltpu.VMEM((40,), jnp.int32),
            pltpu.VMEM((CG,), jnp.float32),
            pltpu.VMEM((GB,), jnp.float32),
            pltpu.VMEM((GB + 16,), jnp.float32),
            pltpu.VMEM((CBP, D), jnp.float32),
            pltpu.VMEM((CBP, H), jnp.float32),
            pltpu.VMEM((SPT, 3 * D + H), jnp.float32),
        ],
    )
    return f(x, local, gate, bnd)


def kernel(x, batch, W_g1, b_g1, W_g2, b_g2, W_l, b_l):
    batch32 = batch.astype(jnp.int32)
    bnd = jnp.searchsorted(batch32, jnp.arange(G + 1, dtype=jnp.int32),
                           side="left").astype(jnp.int32)
    bnd = jnp.concatenate([bnd, jnp.full((23,), N, jnp.int32)])
    gate, local = _dense(x, W_g1, b_g1, W_g2, b_g2, W_l, b_l)
    return _segment_pool(x, local, gate.reshape(N), bnd)
